# baseline (speedup 1.0000x reference)
import functools

import jax
import jax.numpy as jnp
from jax import lax
from jax.experimental import pallas as pl
from jax.experimental.pallas import tpu as pltpu
from jax.experimental.pallas import tpu_sc as plsc

HIDDEN = 4096
IN_SZ = 40
OUT_SZ = 22
FO1 = 2048
FO2 = 11
TAU = 20.0
SPIKE_THRESH = 0.3

E1 = IN_SZ * FO1
E2 = HIDDEN * FO2
NSUB = 16
E1_T = E1 // NSUB
R1_T = E1_T // 128
B1_T = R1_T // 8
CCH = FO1 // 128
H_T = HIDDEN // NSUB
E2_T = E2 // NSUB
R2_T = E2_T // 128


def _snn_body(sp_ref, mt_ref, t1_ref, w1_ref, t2_ref, w2_ref, hl_ref,
              times_ref, pot2_ref,
              sp_v, mt_v, t1_v, w1_v, vals1_v,
              inj1_v, s1_v, t2_v, w2_v, hl_v, t2a_v, vals2_v,
              red_v, out_i_v, out_f_v,
              acc1_sh, acc2_sh, sem, dsem):
    s = lax.axis_index("s")
    decay = jnp.exp(jnp.float32(-1.0 / TAU))
    zero16f = jnp.zeros((16,), jnp.float32)

    stage = [
        pltpu.async_copy(sp_ref, sp_v, dsem),
        pltpu.async_copy(mt_ref, mt_v, dsem),
        pltpu.async_copy(t2_ref.at[pl.ds(s * E2_T, E2_T)], t2_v, dsem),
        pltpu.async_copy(w2_ref.at[pl.ds(s * E2_T, E2_T)], w2_v, dsem),
        pltpu.async_copy(hl_ref, hl_v, dsem),
    ]
    for b in range(B1_T):
        k = s * B1_T + b
        r0 = pl.multiple_of(lax.div(k, CCH) * 8, 8)
        c0 = pl.multiple_of(lax.rem(k, CCH) * 128, 128)
        dst = pl.ds(b * 8, 8)
        stage.append(pltpu.async_copy(
            t1_ref.at[pl.ds(r0, 8), pl.ds(c0, 128)], t1_v.at[dst], dsem))
        stage.append(pltpu.async_copy(
            w1_ref.at[pl.ds(r0, 8), pl.ds(c0, 128)], w1_v.at[dst], dsem))
    for i in range(H_T // 16):
        inj1_v[pl.ds(i * 16, 16)] = zero16f
    pltpu.sync_copy(inj1_v, acc1_sh.at[pl.ds(s * H_T, H_T)])
    pltpu.sync_copy(inj1_v.at[pl.ds(0, 32)], acc2_sh.at[pl.ds(s * 32, 32)])
    for d in stage:
        d.wait()

    plsc.subcore_barrier()

    descs = []
    for r in range(R1_T):
        i_in = lax.div(s * B1_T + r // 8, CCH) * 8 + (r % 8)
        rowv = jnp.full((16,), i_in, jnp.int32)
        sval = plsc.load_gather(sp_v, [rowv])
        sval = sval + sval
        for c in range(8):
            vals1_v[r, pl.ds(c * 16, 16)] = sval * w1_v[r, pl.ds(c * 16, 16)]
        descs.append(
            pltpu.async_copy(vals1_v.at[r], acc1_sh.at[t1_v.at[r]], sem,
                             add=True))
    for d in descs:
        d.wait()

    plsc.subcore_barrier()

    pltpu.sync_copy(acc1_sh.at[pl.ds(s * H_T, H_T)], inj1_v)
    mt = mt_v[pl.ds(0, 16)]
    gate1 = jnp.where(mt >= 1, jnp.full((16,), 1.0, jnp.float32), zero16f)
    for i in range(H_T // 16):
        v = inj1_v[pl.ds(i * 16, 16)]
        s1_v[pl.ds(i * 16, 16)] = jnp.where(v * decay >= SPIKE_THRESH,
                                            gate1, zero16f)

    base2 = s * 32
    descs2 = []
    for j in range(R2_T):
        for c in range(8):
            sl = pl.ds(c * 16, 16)
            fl = pl.ds(j * 128 + c * 16, 16)
            sg = plsc.load_gather(s1_v, [hl_v[j, sl]])
            vals2_v[j, sl] = sg * w2_v[fl]
            t2a_v[j, sl] = t2_v[fl] + base2
        descs2.append(
            pltpu.async_copy(vals2_v.at[j], acc2_sh.at[t2a_v.at[j]], sem,
                             add=True))
    for d in descs2:
        d.wait()

    plsc.subcore_barrier()

    @pl.when(s == 0)
    def _():
        pltpu.sync_copy(acc2_sh, red_v)
        acc_lo = zero16f
        acc_hi = zero16f
        for i in range(NSUB):
            acc_lo = acc_lo + red_v[pl.ds(i * 32, 16)]
            acc_hi = acc_hi + red_v[pl.ds(i * 32 + 16, 16)]
        mtf = mt.astype(jnp.float32)
        live2 = mt >= 2
        scale = jnp.where(live2,
                          jnp.exp(-(mtf - 1.0) * jnp.float32(1.0 / TAU)),
                          zero16f)
        one16 = jnp.full((16,), 1, jnp.int32)
        neg16 = jnp.full((16,), -1, jnp.int32)
        for half, acc in ((0, acc_lo), (1, acc_hi)):
            fired = (acc * decay >= SPIKE_THRESH) & live2
            out_i_v[pl.ds(half * 16, 16)] = jnp.where(fired, one16, neg16)
            out_f_v[pl.ds(half * 16, 16)] = acc * scale
        pltpu.sync_copy(out_i_v, times_ref)
        pltpu.sync_copy(out_f_v, pot2_ref)


@functools.partial(
    pl.kernel,
    out_type=[jax.ShapeDtypeStruct((32,), jnp.int32),
              jax.ShapeDtypeStruct((32,), jnp.float32)],
    mesh=plsc.VectorSubcoreMesh(core_axis_name="c", subcore_axis_name="s",
                                num_cores=1, num_subcores=NSUB),
    compiler_params=pltpu.CompilerParams(needs_layout_passes=False),
    scratch_types=[
        pltpu.VMEM((IN_SZ,), jnp.float32),
        pltpu.VMEM((16,), jnp.int32),
        pltpu.VMEM((R1_T, 128), jnp.int32),
        pltpu.VMEM((R1_T, 128), jnp.float32),
        pltpu.VMEM((R1_T, 128), jnp.float32),
        pltpu.VMEM((H_T,), jnp.float32),
        pltpu.VMEM((H_T,), jnp.float32),
        pltpu.VMEM((E2_T,), jnp.int32),
        pltpu.VMEM((E2_T,), jnp.float32),
        pltpu.VMEM((R2_T, 128), jnp.int32),
        pltpu.VMEM((R2_T, 128), jnp.int32),
        pltpu.VMEM((R2_T, 128), jnp.float32),
        pltpu.VMEM((NSUB * 32,), jnp.float32),
        pltpu.VMEM((32,), jnp.int32),
        pltpu.VMEM((32,), jnp.float32),
        pltpu.VMEM_SHARED((HIDDEN,), jnp.float32),
        pltpu.VMEM_SHARED((NSUB * 32,), jnp.float32),
        pltpu.SemaphoreType.DMA,
        pltpu.SemaphoreType.DMA,
    ],
)
def _snn_sc(*refs):
    _snn_body(*refs)


def kernel(input_spikes, w1, w2, targets1, targets2, max_timesteps):
    mtv = jnp.full((16,), jnp.asarray(max_timesteps, jnp.int32))
    t2 = targets2.reshape(-1)
    w2r = w2.astype(jnp.float32).reshape(-1)
    hl = (jnp.arange(E2_T, dtype=jnp.int32) // FO2).reshape(R2_T, 128)
    times_pad, pot2_pad = _snn_sc(
        input_spikes.astype(jnp.float32), mtv, targets1,
        w1.astype(jnp.float32), t2, w2r, hl)
    return times_pad[:OUT_SZ], pot2_pad[:OUT_SZ]


# speedup vs baseline: 1.1198x; 1.1198x over previous
import functools

import jax
import jax.numpy as jnp
from jax import lax
from jax.experimental import pallas as pl
from jax.experimental.pallas import tpu as pltpu
from jax.experimental.pallas import tpu_sc as plsc

HIDDEN = 4096
IN_SZ = 40
OUT_SZ = 22
FO1 = 2048
FO2 = 11
TAU = 20.0
SPIKE_THRESH = 0.3

E1 = IN_SZ * FO1
E2 = HIDDEN * FO2
NSUB = 16
E1_T = E1 // NSUB
R1_T = E1_T // 128
B1_T = R1_T // 8
CCH = FO1 // 128
H_T = HIDDEN // NSUB
E2_T = E2 // NSUB
R2_T = E2_T // 128


def _snn_body(sp_ref, mt_ref, t1_ref, w1_ref, t2_ref, w2_ref,
              times_ref, pot2_ref,
              sp_v, mt_v, t1_v, w1_v, vals1_v,
              inj1_v, s1_v, t2_v, w2_v, t2a_v, vals2_v,
              red_v, out_i_v, out_f_v,
              acc1_sh, acc2_sh, sem, dsem):
    s = lax.axis_index("s")
    decay = jnp.exp(jnp.float32(-1.0 / TAU))
    zero16f = jnp.zeros((16,), jnp.float32)

    stage = [
        pltpu.async_copy(sp_ref, sp_v, dsem),
        pltpu.async_copy(mt_ref, mt_v, dsem),
        pltpu.async_copy(
            t2_ref.at[pl.ds(0, FO2), pl.ds(s * H_T, H_T)], t2_v, dsem),
        pltpu.async_copy(
            w2_ref.at[pl.ds(0, FO2), pl.ds(s * H_T, H_T)], w2_v, dsem),
    ]
    for b in range(B1_T):
        k = s * B1_T + b
        r0 = pl.multiple_of(lax.div(k, CCH) * 8, 8)
        c0 = pl.multiple_of(lax.rem(k, CCH) * 128, 128)
        dst = pl.ds(b * 8, 8)
        stage.append(pltpu.async_copy(
            t1_ref.at[pl.ds(r0, 8), pl.ds(c0, 128)], t1_v.at[dst], dsem))
        stage.append(pltpu.async_copy(
            w1_ref.at[pl.ds(r0, 8), pl.ds(c0, 128)], w1_v.at[dst], dsem))
    for i in range(H_T // 16):
        inj1_v[pl.ds(i * 16, 16)] = zero16f
    pltpu.sync_copy(inj1_v, acc1_sh.at[pl.ds(s * H_T, H_T)])
    pltpu.sync_copy(inj1_v.at[pl.ds(0, 32)], acc2_sh.at[pl.ds(s * 32, 32)])
    for d in stage:
        d.wait()

    plsc.subcore_barrier()

    descs = []
    for r in range(R1_T):
        i_in = lax.div(s * B1_T + r // 8, CCH) * 8 + (r % 8)
        rowv = jnp.full((16,), i_in, jnp.int32)
        sval = plsc.load_gather(sp_v, [rowv])
        sval = sval + sval
        for c in range(8):
            vals1_v[r, pl.ds(c * 16, 16)] = sval * w1_v[r, pl.ds(c * 16, 16)]
        descs.append(
            pltpu.async_copy(vals1_v.at[r], acc1_sh.at[t1_v.at[r]], sem,
                             add=True))
    for d in descs:
        d.wait()

    plsc.subcore_barrier()

    pltpu.sync_copy(acc1_sh.at[pl.ds(s * H_T, H_T)], inj1_v)
    mt = mt_v[pl.ds(0, 16)]
    gate1 = jnp.where(mt >= 1, jnp.full((16,), 1.0, jnp.float32), zero16f)
    for i in range(H_T // 16):
        v = inj1_v[pl.ds(i * 16, 16)]
        s1_v[pl.ds(i * 16, 16)] = jnp.where(v * decay >= SPIKE_THRESH,
                                            gate1, zero16f)

    base2 = s * 32
    descs2 = []
    for j in range(FO2):
        for c in range(H_T // 16):
            sl = pl.ds(c * 16, 16)
            osl = pl.ds((c % 8) * 16, 16)
            orow = j * 2 + c // 8
            sg = s1_v[sl]
            vals2_v[orow, osl] = sg * w2_v[j, sl]
            t2a_v[orow, osl] = t2_v[j, sl] + base2
            if c % 8 == 7:
                descs2.append(
                    pltpu.async_copy(vals2_v.at[orow],
                                     acc2_sh.at[t2a_v.at[orow]], sem,
                                     add=True))
    for d in descs2:
        d.wait()

    plsc.subcore_barrier()

    @pl.when(s == 0)
    def _():
        pltpu.sync_copy(acc2_sh, red_v)
        acc_lo = zero16f
        acc_hi = zero16f
        for i in range(NSUB):
            acc_lo = acc_lo + red_v[pl.ds(i * 32, 16)]
            acc_hi = acc_hi + red_v[pl.ds(i * 32 + 16, 16)]
        mtf = mt.astype(jnp.float32)
        live2 = mt >= 2
        scale = jnp.where(live2,
                          jnp.exp(-(mtf - 1.0) * jnp.float32(1.0 / TAU)),
                          zero16f)
        one16 = jnp.full((16,), 1, jnp.int32)
        neg16 = jnp.full((16,), -1, jnp.int32)
        for half, acc in ((0, acc_lo), (1, acc_hi)):
            fired = (acc * decay >= SPIKE_THRESH) & live2
            out_i_v[pl.ds(half * 16, 16)] = jnp.where(fired, one16, neg16)
            out_f_v[pl.ds(half * 16, 16)] = acc * scale
        pltpu.sync_copy(out_i_v, times_ref)
        pltpu.sync_copy(out_f_v, pot2_ref)


@functools.partial(
    pl.kernel,
    out_type=[jax.ShapeDtypeStruct((32,), jnp.int32),
              jax.ShapeDtypeStruct((32,), jnp.float32)],
    mesh=plsc.VectorSubcoreMesh(core_axis_name="c", subcore_axis_name="s",
                                num_cores=1, num_subcores=NSUB),
    compiler_params=pltpu.CompilerParams(needs_layout_passes=False),
    scratch_types=[
        pltpu.VMEM((IN_SZ,), jnp.float32),
        pltpu.VMEM((16,), jnp.int32),
        pltpu.VMEM((R1_T, 128), jnp.int32),
        pltpu.VMEM((R1_T, 128), jnp.float32),
        pltpu.VMEM((R1_T, 128), jnp.float32),
        pltpu.VMEM((H_T,), jnp.float32),
        pltpu.VMEM((H_T,), jnp.float32),
        pltpu.VMEM((FO2, H_T), jnp.int32),
        pltpu.VMEM((FO2, H_T), jnp.float32),
        pltpu.VMEM((R2_T, 128), jnp.int32),
        pltpu.VMEM((R2_T, 128), jnp.float32),
        pltpu.VMEM((NSUB * 32,), jnp.float32),
        pltpu.VMEM((32,), jnp.int32),
        pltpu.VMEM((32,), jnp.float32),
        pltpu.VMEM_SHARED((HIDDEN,), jnp.float32),
        pltpu.VMEM_SHARED((NSUB * 32,), jnp.float32),
        pltpu.SemaphoreType.DMA,
        pltpu.SemaphoreType.DMA,
    ],
)
def _snn_sc(*refs):
    _snn_body(*refs)


def kernel(input_spikes, w1, w2, targets1, targets2, max_timesteps):
    mtv = jnp.full((16,), jnp.asarray(max_timesteps, jnp.int32))
    t2 = targets2.T
    w2r = w2.astype(jnp.float32).T
    times_pad, pot2_pad = _snn_sc(
        input_spikes.astype(jnp.float32), mtv, targets1,
        w1.astype(jnp.float32), t2, w2r)
    return times_pad[:OUT_SZ], pot2_pad[:OUT_SZ]


# speedup vs baseline: 1.1989x; 1.0706x over previous
import functools

import jax
import jax.numpy as jnp
from jax import lax
from jax.experimental import pallas as pl
from jax.experimental.pallas import tpu as pltpu
from jax.experimental.pallas import tpu_sc as plsc

HIDDEN = 4096
IN_SZ = 40
OUT_SZ = 22
FO1 = 2048
FO2 = 11
TAU = 20.0
SPIKE_THRESH = 0.3

E1 = IN_SZ * FO1
E2 = HIDDEN * FO2
NSUB = 16
E1_T = E1 // NSUB
R1_T = E1_T // 128
B1_T = R1_T // 8
CCH = FO1 // 128
H_T = HIDDEN // NSUB
E2_T = E2 // NSUB
R2_T = E2_T // 128


def _snn_body(sp_ref, mt_ref, t1_ref, w1_ref, t2_ref, w2_ref,
              times_ref, pot2_ref,
              sp_v, mt_v, t1_v, w1_v, vals1_v,
              inj1_v, s1_v, t2_v, w2_v, t2a_v, vals2_v,
              red_v, out_i_v, out_f_v,
              acc1_sh, acc2_sh, sem, dsem):
    s = lax.axis_index("s")
    decay = jnp.exp(jnp.float32(-1.0 / TAU))
    zero16f = jnp.zeros((16,), jnp.float32)

    stage = [
        pltpu.async_copy(sp_ref, sp_v, dsem),
        pltpu.async_copy(mt_ref, mt_v, dsem),
        pltpu.async_copy(
            t2_ref.at[pl.ds(0, FO2), pl.ds(s * H_T, H_T)], t2_v, dsem),
        pltpu.async_copy(
            w2_ref.at[pl.ds(0, FO2), pl.ds(s * H_T, H_T)], w2_v, dsem),
    ]
    for b in range(B1_T):
        k = s * B1_T + b
        r0 = pl.multiple_of(lax.div(k, CCH) * 8, 8)
        c0 = pl.multiple_of(lax.rem(k, CCH) * 128, 128)
        dst = pl.ds(b * 8, 8)
        stage.append(pltpu.async_copy(
            t1_ref.at[pl.ds(r0, 8), pl.ds(c0, 128)], t1_v.at[dst], dsem))
        stage.append(pltpu.async_copy(
            w1_ref.at[pl.ds(r0, 8), pl.ds(c0, 128)], w1_v.at[dst], dsem))
    for i in range(H_T // 16):
        inj1_v[pl.ds(i * 16, 16)] = zero16f
    pltpu.sync_copy(inj1_v, acc1_sh.at[pl.ds(s * H_T, H_T)])
    pltpu.sync_copy(inj1_v.at[pl.ds(0, 32)], acc2_sh.at[pl.ds(s * 32, 32)])
    for d in stage:
        d.wait()

    plsc.subcore_barrier()

    def l1_row(r, carry):
        i_in = lax.div(s * B1_T + lax.div(r, 8), CCH) * 8 + lax.rem(r, 8)
        rowv = jnp.full((16,), i_in, jnp.int32)
        sval = plsc.load_gather(sp_v, [rowv])
        sval = sval + sval
        for c in range(8):
            vals1_v[r, pl.ds(c * 16, 16)] = sval * w1_v[r, pl.ds(c * 16, 16)]
        pltpu.async_copy(vals1_v.at[r], acc1_sh.at[t1_v.at[r]], sem,
                         add=True)
        return carry

    lax.fori_loop(0, R1_T, l1_row, 0)
    pltpu.make_async_copy(
        w1_ref.at[pl.ds(0, R1_T), pl.ds(0, 128)], vals1_v, sem).wait()

    plsc.subcore_barrier()

    pltpu.sync_copy(acc1_sh.at[pl.ds(s * H_T, H_T)], inj1_v)
    mt = mt_v[pl.ds(0, 16)]
    gate1 = jnp.where(mt >= 1, jnp.full((16,), 1.0, jnp.float32), zero16f)
    for i in range(H_T // 16):
        v = inj1_v[pl.ds(i * 16, 16)]
        s1_v[pl.ds(i * 16, 16)] = jnp.where(v * decay >= SPIKE_THRESH,
                                            gate1, zero16f)

    base2 = s * 32

    def l2_row(j, carry):
        for c in range(H_T // 16):
            sl = pl.ds(c * 16, 16)
            osl = pl.ds((c % 8) * 16, 16)
            orow = j * 2 + c // 8
            sg = s1_v[sl]
            vals2_v[orow, osl] = sg * w2_v[j, sl]
            t2a_v[orow, osl] = t2_v[j, sl] + base2
            if c % 8 == 7:
                pltpu.async_copy(vals2_v.at[orow],
                                 acc2_sh.at[t2a_v.at[orow]], sem,
                                 add=True)
        return carry

    lax.fori_loop(0, FO2, l2_row, 0)
    for j in range(R2_T):
        pltpu.make_async_copy(vals2_v.at[j], acc2_sh.at[t2a_v.at[j]],
                              sem).wait()

    plsc.subcore_barrier()

    @pl.when(s == 0)
    def _():
        pltpu.sync_copy(acc2_sh, red_v)
        acc_lo = zero16f
        acc_hi = zero16f
        for i in range(NSUB):
            acc_lo = acc_lo + red_v[pl.ds(i * 32, 16)]
            acc_hi = acc_hi + red_v[pl.ds(i * 32 + 16, 16)]
        mtf = mt.astype(jnp.float32)
        live2 = mt >= 2
        scale = jnp.where(live2,
                          jnp.exp(-(mtf - 1.0) * jnp.float32(1.0 / TAU)),
                          zero16f)
        one16 = jnp.full((16,), 1, jnp.int32)
        neg16 = jnp.full((16,), -1, jnp.int32)
        for half, acc in ((0, acc_lo), (1, acc_hi)):
            fired = (acc * decay >= SPIKE_THRESH) & live2
            out_i_v[pl.ds(half * 16, 16)] = jnp.where(fired, one16, neg16)
            out_f_v[pl.ds(half * 16, 16)] = acc * scale
        pltpu.sync_copy(out_i_v, times_ref)
        pltpu.sync_copy(out_f_v, pot2_ref)


@functools.partial(
    pl.kernel,
    out_type=[jax.ShapeDtypeStruct((32,), jnp.int32),
              jax.ShapeDtypeStruct((32,), jnp.float32)],
    mesh=plsc.VectorSubcoreMesh(core_axis_name="c", subcore_axis_name="s",
                                num_cores=1, num_subcores=NSUB),
    compiler_params=pltpu.CompilerParams(needs_layout_passes=False),
    scratch_types=[
        pltpu.VMEM((IN_SZ,), jnp.float32),
        pltpu.VMEM((16,), jnp.int32),
        pltpu.VMEM((R1_T, 128), jnp.int32),
        pltpu.VMEM((R1_T, 128), jnp.float32),
        pltpu.VMEM((R1_T, 128), jnp.float32),
        pltpu.VMEM((H_T,), jnp.float32),
        pltpu.VMEM((H_T,), jnp.float32),
        pltpu.VMEM((FO2, H_T), jnp.int32),
        pltpu.VMEM((FO2, H_T), jnp.float32),
        pltpu.VMEM((R2_T, 128), jnp.int32),
        pltpu.VMEM((R2_T, 128), jnp.float32),
        pltpu.VMEM((NSUB * 32,), jnp.float32),
        pltpu.VMEM((32,), jnp.int32),
        pltpu.VMEM((32,), jnp.float32),
        pltpu.VMEM_SHARED((HIDDEN,), jnp.float32),
        pltpu.VMEM_SHARED((NSUB * 32,), jnp.float32),
        pltpu.SemaphoreType.DMA,
        pltpu.SemaphoreType.DMA,
    ],
)
def _snn_sc(*refs):
    _snn_body(*refs)


def kernel(input_spikes, w1, w2, targets1, targets2, max_timesteps):
    mtv = jnp.full((16,), jnp.asarray(max_timesteps, jnp.int32))
    t2 = targets2.T
    w2r = w2.astype(jnp.float32).T
    times_pad, pot2_pad = _snn_sc(
        input_spikes.astype(jnp.float32), mtv, targets1,
        w1.astype(jnp.float32), t2, w2r)
    return times_pad[:OUT_SZ], pot2_pad[:OUT_SZ]


# speedup vs baseline: 1.2289x; 1.0250x over previous
import functools

import jax
import jax.numpy as jnp
from jax import lax
from jax.experimental import pallas as pl
from jax.experimental.pallas import tpu as pltpu
from jax.experimental.pallas import tpu_sc as plsc

HIDDEN = 4096
IN_SZ = 40
OUT_SZ = 22
FO1 = 2048
FO2 = 11
TAU = 20.0
SPIKE_THRESH = 0.3

E1 = IN_SZ * FO1
E2 = HIDDEN * FO2
NSUB = 16
E1_T = E1 // NSUB
R1_T = E1_T // 128
B1_T = R1_T // 8
CCH = FO1 // 128
H_T = HIDDEN // NSUB
E2_T = E2 // NSUB
R2_T = E2_T // 128


def _snn_body(sp_ref, mt_ref, t1_ref, w1_ref, t2_ref, w2_ref,
              times_ref, pot2_ref,
              sp_v, mt_v, t1_v, w1_v, vals1_v,
              inj1_v, s1_v, t2_v, w2_v, t2a_v, vals2_v,
              red_v, out_i_v, out_f_v,
              acc1_sh, acc2_sh, sem, dsem):
    s = lax.axis_index("s")
    decay = jnp.exp(jnp.float32(-1.0 / TAU))
    zero16f = jnp.zeros((16,), jnp.float32)

    stage = [
        pltpu.async_copy(sp_ref, sp_v, dsem),
        pltpu.async_copy(mt_ref, mt_v, dsem),
        pltpu.async_copy(
            t2_ref.at[pl.ds(0, FO2), pl.ds(s * H_T, H_T)], t2_v, dsem),
        pltpu.async_copy(
            w2_ref.at[pl.ds(0, FO2), pl.ds(s * H_T, H_T)], w2_v, dsem),
    ]
    for b in range(B1_T):
        k = s * B1_T + b
        r0 = pl.multiple_of(lax.div(k, CCH) * 8, 8)
        c0 = pl.multiple_of(lax.rem(k, CCH) * 128, 128)
        dst = pl.ds(b * 8, 8)
        stage.append(pltpu.async_copy(
            t1_ref.at[pl.ds(r0, 8), pl.ds(c0, 128)], t1_v.at[dst], dsem))
        stage.append(pltpu.async_copy(
            w1_ref.at[pl.ds(r0, 8), pl.ds(c0, 128)], w1_v.at[dst], dsem))
    for i in range(H_T // 16):
        inj1_v[pl.ds(i * 16, 16)] = zero16f
    pltpu.sync_copy(inj1_v, acc1_sh.at[pl.ds(s * H_T, H_T)])
    pltpu.sync_copy(inj1_v.at[pl.ds(0, 32)], acc2_sh.at[pl.ds(s * 32, 32)])
    for d in stage:
        d.wait()

    plsc.subcore_barrier()

    def l1_row(h, carry):
        for u in range(2):
            r = h * 2 + u
            i_in = (lax.div(s * B1_T + lax.div(r, 8), CCH) * 8
                    + lax.rem(r, 8))
            rowv = jnp.full((16,), i_in, jnp.int32)
            sval = plsc.load_gather(sp_v, [rowv])
            sval = sval + sval
            for c in range(8):
                vals1_v[r, pl.ds(c * 16, 16)] = (
                    sval * w1_v[r, pl.ds(c * 16, 16)])
            pltpu.async_copy(vals1_v.at[r], acc1_sh.at[t1_v.at[r]], sem,
                             add=True)
        return carry

    lax.fori_loop(0, R1_T // 2, l1_row, 0)

    base2 = s * 32

    def l2_idx(j, carry):
        for c in range(H_T // 16):
            t2a_v[j * 2 + c // 8, pl.ds((c % 8) * 16, 16)] = (
                t2_v[j, pl.ds(c * 16, 16)] + base2)
        return carry

    lax.fori_loop(0, FO2, l2_idx, 0)

    pltpu.make_async_copy(
        w1_ref.at[pl.ds(0, R1_T), pl.ds(0, 128)], vals1_v, sem).wait()

    plsc.subcore_barrier()

    pltpu.sync_copy(acc1_sh.at[pl.ds(s * H_T, H_T)], inj1_v)
    mt = mt_v[pl.ds(0, 16)]
    gate1 = jnp.where(mt >= 1, jnp.full((16,), 1.0, jnp.float32), zero16f)
    for i in range(H_T // 16):
        v = inj1_v[pl.ds(i * 16, 16)]
        s1_v[pl.ds(i * 16, 16)] = jnp.where(v * decay >= SPIKE_THRESH,
                                            gate1, zero16f)

    def l2_row(j, carry):
        for c in range(H_T // 16):
            sl = pl.ds(c * 16, 16)
            osl = pl.ds((c % 8) * 16, 16)
            orow = j * 2 + c // 8
            vals2_v[orow, osl] = s1_v[sl] * w2_v[j, sl]
            if c % 8 == 7:
                pltpu.async_copy(vals2_v.at[orow],
                                 acc2_sh.at[t2a_v.at[orow]], sem,
                                 add=True)
        return carry

    lax.fori_loop(0, FO2, l2_row, 0)
    for j in range(R2_T):
        pltpu.make_async_copy(vals2_v.at[j], acc2_sh.at[t2a_v.at[j]],
                              sem).wait()

    plsc.subcore_barrier()

    @pl.when(s == 0)
    def _():
        pltpu.sync_copy(acc2_sh, red_v)
        acc_lo = zero16f
        acc_hi = zero16f
        for i in range(NSUB):
            acc_lo = acc_lo + red_v[pl.ds(i * 32, 16)]
            acc_hi = acc_hi + red_v[pl.ds(i * 32 + 16, 16)]
        mtf = mt.astype(jnp.float32)
        live2 = mt >= 2
        scale = jnp.where(live2,
                          jnp.exp(-(mtf - 1.0) * jnp.float32(1.0 / TAU)),
                          zero16f)
        one16 = jnp.full((16,), 1, jnp.int32)
        neg16 = jnp.full((16,), -1, jnp.int32)
        for half, acc in ((0, acc_lo), (1, acc_hi)):
            fired = (acc * decay >= SPIKE_THRESH) & live2
            out_i_v[pl.ds(half * 16, 16)] = jnp.where(fired, one16, neg16)
            out_f_v[pl.ds(half * 16, 16)] = acc * scale
        pltpu.sync_copy(out_i_v, times_ref)
        pltpu.sync_copy(out_f_v, pot2_ref)


@functools.partial(
    pl.kernel,
    out_type=[jax.ShapeDtypeStruct((32,), jnp.int32),
              jax.ShapeDtypeStruct((32,), jnp.float32)],
    mesh=plsc.VectorSubcoreMesh(core_axis_name="c", subcore_axis_name="s",
                                num_cores=1, num_subcores=NSUB),
    compiler_params=pltpu.CompilerParams(needs_layout_passes=False),
    scratch_types=[
        pltpu.VMEM((IN_SZ,), jnp.float32),
        pltpu.VMEM((16,), jnp.int32),
        pltpu.VMEM((R1_T, 128), jnp.int32),
        pltpu.VMEM((R1_T, 128), jnp.float32),
        pltpu.VMEM((R1_T, 128), jnp.float32),
        pltpu.VMEM((H_T,), jnp.float32),
        pltpu.VMEM((H_T,), jnp.float32),
        pltpu.VMEM((FO2, H_T), jnp.int32),
        pltpu.VMEM((FO2, H_T), jnp.float32),
        pltpu.VMEM((R2_T, 128), jnp.int32),
        pltpu.VMEM((R2_T, 128), jnp.float32),
        pltpu.VMEM((NSUB * 32,), jnp.float32),
        pltpu.VMEM((32,), jnp.int32),
        pltpu.VMEM((32,), jnp.float32),
        pltpu.VMEM_SHARED((HIDDEN,), jnp.float32),
        pltpu.VMEM_SHARED((NSUB * 32,), jnp.float32),
        pltpu.SemaphoreType.DMA,
        pltpu.SemaphoreType.DMA,
    ],
)
def _snn_sc(*refs):
    _snn_body(*refs)


def kernel(input_spikes, w1, w2, targets1, targets2, max_timesteps):
    mtv = jnp.full((16,), jnp.asarray(max_timesteps, jnp.int32))
    t2 = targets2.T
    w2r = w2.astype(jnp.float32).T
    times_pad, pot2_pad = _snn_sc(
        input_spikes.astype(jnp.float32), mtv, targets1,
        w1.astype(jnp.float32), t2, w2r)
    return times_pad[:OUT_SZ], pot2_pad[:OUT_SZ]


# speedup vs baseline: 1.2301x; 1.0010x over previous
import functools

import jax
import jax.numpy as jnp
from jax import lax
from jax.experimental import pallas as pl
from jax.experimental.pallas import tpu as pltpu
from jax.experimental.pallas import tpu_sc as plsc

HIDDEN = 4096
IN_SZ = 40
OUT_SZ = 22
FO1 = 2048
FO2 = 11
TAU = 20.0
SPIKE_THRESH = 0.3

E1 = IN_SZ * FO1
E2 = HIDDEN * FO2
NSUB = 16
E1_T = E1 // NSUB
R1_T = E1_T // 128
B1_T = R1_T // 8
CCH = FO1 // 128
H_T = HIDDEN // NSUB
E2_T = E2 // NSUB
R2_T = E2_T // 128


def _snn_body(sp_ref, mt_ref, t1_ref, w1_ref, t2_ref, w2_ref, dum_ref,
              times_ref, pot2_ref,
              sp_v, mt_v, t1_v, w1_v, vals1_v,
              inj1_v, s1_v, t2_v, w2_v, t2a_v, vals2_v,
              red_v, out_i_v, out_f_v,
              acc1_sh, acc2_sh, sem, dsem):
    s = lax.axis_index("s")
    decay = jnp.exp(jnp.float32(-1.0 / TAU))
    zero16f = jnp.zeros((16,), jnp.float32)

    stage = [
        pltpu.async_copy(sp_ref, sp_v, dsem),
        pltpu.async_copy(mt_ref, mt_v, dsem),
        pltpu.async_copy(
            t2_ref.at[pl.ds(0, FO2), pl.ds(s * H_T, H_T)], t2_v, dsem),
        pltpu.async_copy(
            w2_ref.at[pl.ds(0, FO2), pl.ds(s * H_T, H_T)], w2_v, dsem),
    ]
    for b in range(B1_T):
        k = s * B1_T + b
        r0 = pl.multiple_of(lax.div(k, CCH) * 8, 8)
        c0 = pl.multiple_of(lax.rem(k, CCH) * 128, 128)
        dst = pl.ds(b * 8, 8)
        stage.append(pltpu.async_copy(
            t1_ref.at[pl.ds(r0, 8), pl.ds(c0, 128)], t1_v.at[dst], dsem))
        stage.append(pltpu.async_copy(
            w1_ref.at[pl.ds(r0, 8), pl.ds(c0, 128)], w1_v.at[dst], dsem))
    def zloop(i, carry):
        inj1_v[pl.ds(i * 16, 16)] = zero16f
        return carry

    lax.fori_loop(0, H_T // 16, zloop, 0)
    pltpu.sync_copy(inj1_v, acc1_sh.at[pl.ds(s * H_T, H_T)])
    pltpu.sync_copy(inj1_v.at[pl.ds(0, 32)], acc2_sh.at[pl.ds(s * 32, 32)])
    for d in stage:
        d.wait()

    plsc.subcore_barrier()

    def l1_row(h, carry):
        for u in range(2):
            r = h * 2 + u
            i_in = (lax.div(s * B1_T + lax.div(r, 8), CCH) * 8
                    + lax.rem(r, 8))
            rowv = jnp.full((16,), i_in, jnp.int32)
            sval = plsc.load_gather(sp_v, [rowv])
            sval = sval + sval
            for c in range(8):
                vals1_v[r, pl.ds(c * 16, 16)] = (
                    sval * w1_v[r, pl.ds(c * 16, 16)])
            pltpu.async_copy(vals1_v.at[r], acc1_sh.at[t1_v.at[r]], sem,
                             add=True)
        return carry

    lax.fori_loop(0, R1_T // 2, l1_row, 0)

    base2 = s * 32

    def l2_idx(j, carry):
        for c in range(H_T // 16):
            t2a_v[j * 2 + c // 8, pl.ds((c % 8) * 16, 16)] = (
                t2_v[j, pl.ds(c * 16, 16)] + base2)
        return carry

    lax.fori_loop(0, FO2, l2_idx, 0)

    pltpu.make_async_copy(
        w1_ref.at[pl.ds(0, R1_T), pl.ds(0, 128)], vals1_v, sem).wait()

    plsc.subcore_barrier()

    pltpu.sync_copy(acc1_sh.at[pl.ds(s * H_T, H_T)], inj1_v)
    mt = mt_v[pl.ds(0, 16)]
    gate1 = jnp.where(mt >= 1, jnp.full((16,), 1.0, jnp.float32), zero16f)

    def thloop(i, carry):
        v = inj1_v[pl.ds(i * 16, 16)]
        s1_v[pl.ds(i * 16, 16)] = jnp.where(v * decay >= SPIKE_THRESH,
                                            gate1, zero16f)
        return carry

    lax.fori_loop(0, H_T // 16, thloop, 0)

    def l2_row(j, carry):
        for c in range(H_T // 16):
            sl = pl.ds(c * 16, 16)
            vals2_v[pl.ds(j * 256 + c * 16, 16)] = s1_v[sl] * w2_v[j, sl]
            if c % 8 == 7:
                orow = j * 2 + c // 8
                pltpu.async_copy(vals2_v.at[pl.ds(orow * 128, 128)],
                                 acc2_sh.at[t2a_v.at[orow]], sem,
                                 add=True)
        return carry

    lax.fori_loop(0, FO2, l2_row, 0)
    pltpu.make_async_copy(dum_ref, vals2_v, sem).wait()

    plsc.subcore_barrier()

    @pl.when(s == 0)
    def _():
        pltpu.sync_copy(acc2_sh, red_v)

        def redloop(i, carry):
            lo, hi = carry
            return (lo + red_v[pl.ds(i * 32, 16)],
                    hi + red_v[pl.ds(i * 32 + 16, 16)])

        acc_lo, acc_hi = lax.fori_loop(0, NSUB, redloop,
                                       (zero16f, zero16f))
        mtf = mt.astype(jnp.float32)
        live2 = mt >= 2
        scale = jnp.where(live2,
                          jnp.exp(-(mtf - 1.0) * jnp.float32(1.0 / TAU)),
                          zero16f)
        one16 = jnp.full((16,), 1, jnp.int32)
        neg16 = jnp.full((16,), -1, jnp.int32)
        for half, acc in ((0, acc_lo), (1, acc_hi)):
            fired = (acc * decay >= SPIKE_THRESH) & live2
            out_i_v[pl.ds(half * 16, 16)] = jnp.where(fired, one16, neg16)
            out_f_v[pl.ds(half * 16, 16)] = acc * scale
        pltpu.sync_copy(out_i_v, times_ref)
        pltpu.sync_copy(out_f_v, pot2_ref)


@functools.partial(
    pl.kernel,
    out_type=[jax.ShapeDtypeStruct((32,), jnp.int32),
              jax.ShapeDtypeStruct((32,), jnp.float32)],
    mesh=plsc.VectorSubcoreMesh(core_axis_name="c", subcore_axis_name="s",
                                num_cores=1, num_subcores=NSUB),
    compiler_params=pltpu.CompilerParams(needs_layout_passes=False),
    scratch_types=[
        pltpu.VMEM((IN_SZ,), jnp.float32),
        pltpu.VMEM((16,), jnp.int32),
        pltpu.VMEM((R1_T, 128), jnp.int32),
        pltpu.VMEM((R1_T, 128), jnp.float32),
        pltpu.VMEM((R1_T, 128), jnp.float32),
        pltpu.VMEM((H_T,), jnp.float32),
        pltpu.VMEM((H_T,), jnp.float32),
        pltpu.VMEM((FO2, H_T), jnp.int32),
        pltpu.VMEM((FO2, H_T), jnp.float32),
        pltpu.VMEM((R2_T, 128), jnp.int32),
        pltpu.VMEM((E2_T,), jnp.float32),
        pltpu.VMEM((NSUB * 32,), jnp.float32),
        pltpu.VMEM((32,), jnp.int32),
        pltpu.VMEM((32,), jnp.float32),
        pltpu.VMEM_SHARED((HIDDEN,), jnp.float32),
        pltpu.VMEM_SHARED((NSUB * 32,), jnp.float32),
        pltpu.SemaphoreType.DMA,
        pltpu.SemaphoreType.DMA,
    ],
)
def _snn_sc(*refs):
    _snn_body(*refs)


def kernel(input_spikes, w1, w2, targets1, targets2, max_timesteps):
    mtv = jnp.full((16,), jnp.asarray(max_timesteps, jnp.int32))
    t2 = targets2.T
    w2r = w2.astype(jnp.float32).T
    dummy = jnp.zeros((E2_T,), jnp.float32)
    times_pad, pot2_pad = _snn_sc(
        input_spikes.astype(jnp.float32), mtv, targets1,
        w1.astype(jnp.float32), t2, w2r, dummy)
    return times_pad[:OUT_SZ], pot2_pad[:OUT_SZ]
